# baseline (speedup 1.0000x reference)
import jax
import jax.numpy as jnp
import numpy as np
from jax import lax
from jax.experimental import pallas as pl
from jax.experimental.pallas import tpu as pltpu
from jax.experimental.pallas import tpu_sc as plsc

_N = 10000
_NP = 10240
_SLICE = _NP // 16
_NSUB = 16
_E = 320000
_EPT = _E // _NSUB
_EPAD = 20480
_PAD = _NP - 1
_ALPHA = 0.85
_MAXIT = 100
_THRESH = np.float32(_N * 1e-06)
_V0 = np.float32(1.0 / _N)
_ADDEND = np.float32(np.float32(1.0 / _N) * np.float32(1.0 - _ALPHA))
_LN = 16


def _body(rows_hbm, cols_hbm, out_hbm, rows_v, cols_v, ev,
          vsl, usl, msl, ainv, addend, zbuf, errw, errall,
          u_sh, mv_sh, err_sh):
    s = lax.axis_index("s")
    c = lax.axis_index("c")
    base = s * _SLICE
    ebase = s * _EPAD

    pltpu.sync_copy(rows_hbm.at[pl.ds(ebase, _EPAD)], rows_v)
    pltpu.sync_copy(cols_hbm.at[pl.ds(ebase, _EPAD)], cols_v)

    lane = lax.iota(jnp.int32, 16)

    def initk(k, carry):
        gidx = base + (k * _LN) + lane
        m = gidx < _N
        vsl[pl.ds(k * 16, 16)] = jnp.where(m, _V0, np.float32(0.0))
        addend[pl.ds(k * 16, 16)] = jnp.where(m, _ADDEND, np.float32(0.0))
        zbuf[pl.ds(k * 16, 16)] = jnp.zeros((16,), jnp.float32)
        usl[pl.ds(k * 16, 16)] = jnp.ones((16,), jnp.float32)
        return carry

    lax.fori_loop(0, _SLICE // _LN, initk, 0)

    pltpu.sync_copy(usl, u_sh.at[pl.ds(base, _SLICE)])
    pltpu.sync_copy(zbuf, mv_sh.at[pl.ds(base, _SLICE)])
    plsc.subcore_barrier()
    pltpu.sync_copy(u_sh.at[cols_v], ev)
    pltpu.sync_copy(ev, mv_sh.at[cols_v], add=True)
    plsc.subcore_barrier()
    pltpu.sync_copy(mv_sh.at[pl.ds(base, _SLICE)], msl)
    pltpu.sync_copy(zbuf, mv_sh.at[pl.ds(base, _SLICE)])
    def ainvk(k, carry):
        d = msl[pl.ds(k * 16, 16)]
        ainv[pl.ds(k * 16, 16)] = jnp.where(
            d > np.float32(0.0), np.float32(_ALPHA) / d, np.float32(0.0))
        return carry

    lax.fori_loop(0, _SLICE // _LN, ainvk, 0)

    errw[...] = jnp.ones((16,), jnp.float32)

    def itbody(i, _):
        t = errw[...]
        done = t[0] < _THRESH

        @pl.when(jnp.logical_not(done))
        def _():
            def uk(k, carry):
                usl[pl.ds(k * 16, 16)] = (
                    vsl[pl.ds(k * 16, 16)] * ainv[pl.ds(k * 16, 16)])
                return carry

            lax.fori_loop(0, _SLICE // _LN, uk, 0)
            pltpu.sync_copy(usl, u_sh.at[pl.ds(base, _SLICE)])
            plsc.subcore_barrier()

            pltpu.sync_copy(u_sh.at[cols_v], ev)
            pltpu.sync_copy(ev, mv_sh.at[rows_v], add=True)
            plsc.subcore_barrier()

            pltpu.sync_copy(mv_sh.at[pl.ds(base, _SLICE)], msl)
            pltpu.sync_copy(zbuf, mv_sh.at[pl.ds(base, _SLICE)])

            def vk(k, errv):
                vn = msl[pl.ds(k * 16, 16)] + addend[pl.ds(k * 16, 16)]
                errv = errv + jnp.abs(vn - vsl[pl.ds(k * 16, 16)])
                vsl[pl.ds(k * 16, 16)] = vn
                return errv

            errv = lax.fori_loop(
                0, _SLICE // _LN, vk, jnp.zeros((16,), jnp.float32))
            errw[...] = errv
            pltpu.sync_copy(errw, err_sh.at[s])
            plsc.subcore_barrier()

            pltpu.sync_copy(err_sh, errall)

            def sumk(k, acc):
                return acc + errall[k]

            tot = lax.fori_loop(
                0, _NSUB, sumk, jnp.zeros((16,), jnp.float32))
            total = np.float32(0.0)
            for j in range(_LN):
                total = total + tot[j]
            errw[...] = jnp.full((16,), np.float32(1.0)) * total

        return 0

    lax.fori_loop(0, _MAXIT, itbody, 0)

    @pl.when(c == jnp.int32(0))
    def _():
        pltpu.sync_copy(vsl, out_hbm.at[pl.ds(base, _SLICE)])


_pr_call = pl.kernel(
    _body,
    out_type=jax.ShapeDtypeStruct((_NP,), jnp.float32),
    mesh=plsc.VectorSubcoreMesh(
        core_axis_name="c", subcore_axis_name="s",
        num_cores=2, num_subcores=_NSUB),
    scratch_types=[
        pltpu.VMEM((_EPAD,), jnp.int32),
        pltpu.VMEM((_EPAD,), jnp.int32),
        pltpu.VMEM((_EPAD,), jnp.float32),
        pltpu.VMEM((_SLICE,), jnp.float32),
        pltpu.VMEM((_SLICE,), jnp.float32),
        pltpu.VMEM((_SLICE,), jnp.float32),
        pltpu.VMEM((_SLICE,), jnp.float32),
        pltpu.VMEM((_SLICE,), jnp.float32),
        pltpu.VMEM((_SLICE,), jnp.float32),
        pltpu.VMEM((16,), jnp.float32),
        pltpu.VMEM((16, 16), jnp.float32),
        pltpu.VMEM_SHARED((_NP,), jnp.float32),
        pltpu.VMEM_SHARED((_NP,), jnp.float32),
        pltpu.VMEM_SHARED((16, 16), jnp.float32),
    ],
)


@jax.jit
def kernel(x, edge_index):
    del x
    rows = edge_index[0]
    cols = edge_index[1]
    pad = jnp.full((_NSUB, _EPAD - _EPT), _PAD, jnp.int32)
    rows1d = jnp.concatenate(
        [rows.reshape(_NSUB, _EPT), pad], axis=1).reshape(_NSUB * _EPAD)
    cols1d = jnp.concatenate(
        [cols.reshape(_NSUB, _EPT), pad], axis=1).reshape(_NSUB * _EPAD)
    out = _pr_call(rows1d, cols1d)
    return out[:_N]


# speedup vs baseline: 1.5973x; 1.5973x over previous
import jax
import jax.numpy as jnp
import numpy as np
from jax import lax
from jax.experimental import pallas as pl
from jax.experimental.pallas import tpu as pltpu
from jax.experimental.pallas import tpu_sc as plsc

_N = 10000
_NP = 10240
_SLICE = _NP // 16
_NSUB = 16
_E = 320000
_EPT = _E // _NSUB
_ALPHA = 0.85
_MAXIT = 100
_THRESH = np.float32(_N * 1e-06)
_V0 = np.float32(1.0 / _N)
_ADDEND = np.float32(np.float32(1.0 / _N) * np.float32(1.0 - _ALPHA))
_LN = 16


def _body(rows_hbm, cols_hbm, out_hbm, rows_v, cols_v, ev,
          vsl, usl, msl, ainv, addend, zbuf, errw, errall,
          u_sh, mv_sh, err_sh):
    s = lax.axis_index("s")
    c = lax.axis_index("c")
    base = s * _SLICE
    ebase = s * _EPT

    pltpu.sync_copy(rows_hbm.at[pl.ds(ebase, _EPT)], rows_v)
    pltpu.sync_copy(cols_hbm.at[pl.ds(ebase, _EPT)], cols_v)

    lane = lax.iota(jnp.int32, 16)

    def initk(k, carry):
        gidx = base + (k * _LN) + lane
        m = gidx < _N
        vsl[pl.ds(k * 16, 16)] = jnp.where(m, _V0, np.float32(0.0))
        addend[pl.ds(k * 16, 16)] = jnp.where(m, _ADDEND, np.float32(0.0))
        zbuf[pl.ds(k * 16, 16)] = jnp.zeros((16,), jnp.float32)
        return carry

    lax.fori_loop(0, _SLICE // _LN, initk, 0)

    def onesk(k, carry):
        ev[pl.ds(k * 16, 16)] = jnp.ones((16,), jnp.float32)
        return carry

    lax.fori_loop(0, _EPT // _LN, onesk, 0)

    pltpu.sync_copy(zbuf, mv_sh.at[pl.ds(base, _SLICE)])
    plsc.subcore_barrier()
    pltpu.sync_copy(ev, mv_sh.at[cols_v], add=True)
    plsc.subcore_barrier()
    pltpu.sync_copy(mv_sh.at[pl.ds(base, _SLICE)], msl)
    pltpu.sync_copy(zbuf, mv_sh.at[pl.ds(base, _SLICE)])
    def ainvk(k, carry):
        d = msl[pl.ds(k * 16, 16)]
        ainv[pl.ds(k * 16, 16)] = jnp.where(
            d > np.float32(0.0), np.float32(_ALPHA) / d, np.float32(0.0))
        return carry

    lax.fori_loop(0, _SLICE // _LN, ainvk, 0)

    errw[...] = jnp.ones((16,), jnp.float32)

    def itbody(i, _):
        t = errw[...]
        done = t[0] < _THRESH

        @pl.when(jnp.logical_not(done))
        def _():
            def uk(k, carry):
                usl[pl.ds(k * 16, 16)] = (
                    vsl[pl.ds(k * 16, 16)] * ainv[pl.ds(k * 16, 16)])
                return carry

            lax.fori_loop(0, _SLICE // _LN, uk, 0)
            pltpu.sync_copy(usl, u_sh.at[pl.ds(base, _SLICE)])
            plsc.subcore_barrier()

            pltpu.sync_copy(u_sh.at[cols_v], ev)
            pltpu.sync_copy(ev, mv_sh.at[rows_v], add=True)
            plsc.subcore_barrier()

            pltpu.sync_copy(mv_sh.at[pl.ds(base, _SLICE)], msl)
            pltpu.sync_copy(zbuf, mv_sh.at[pl.ds(base, _SLICE)])

            def vk(k, errv):
                vn = msl[pl.ds(k * 16, 16)] + addend[pl.ds(k * 16, 16)]
                errv = errv + jnp.abs(vn - vsl[pl.ds(k * 16, 16)])
                vsl[pl.ds(k * 16, 16)] = vn
                return errv

            errv = lax.fori_loop(
                0, _SLICE // _LN, vk, jnp.zeros((16,), jnp.float32))
            errw[...] = errv
            pltpu.sync_copy(errw, err_sh.at[s])
            plsc.subcore_barrier()

            pltpu.sync_copy(err_sh, errall)

            def sumk(k, acc):
                return acc + errall[k]

            tot = lax.fori_loop(
                0, _NSUB, sumk, jnp.zeros((16,), jnp.float32))
            total = np.float32(0.0)
            for j in range(_LN):
                total = total + tot[j]
            errw[...] = jnp.full((16,), np.float32(1.0)) * total

        return 0

    lax.fori_loop(0, _MAXIT, itbody, 0)

    @pl.when(c == jnp.int32(0))
    def _():
        pltpu.sync_copy(vsl, out_hbm.at[pl.ds(base, _SLICE)])


_pr_call = pl.kernel(
    _body,
    out_type=jax.ShapeDtypeStruct((_NP,), jnp.float32),
    mesh=plsc.VectorSubcoreMesh(
        core_axis_name="c", subcore_axis_name="s",
        num_cores=2, num_subcores=_NSUB),
    scratch_types=[
        pltpu.VMEM((_EPT,), jnp.int32),
        pltpu.VMEM((_EPT,), jnp.int32),
        pltpu.VMEM((_EPT,), jnp.float32),
        pltpu.VMEM((_SLICE,), jnp.float32),
        pltpu.VMEM((_SLICE,), jnp.float32),
        pltpu.VMEM((_SLICE,), jnp.float32),
        pltpu.VMEM((_SLICE,), jnp.float32),
        pltpu.VMEM((_SLICE,), jnp.float32),
        pltpu.VMEM((_SLICE,), jnp.float32),
        pltpu.VMEM((16,), jnp.float32),
        pltpu.VMEM((16, 16), jnp.float32),
        pltpu.VMEM_SHARED((_NP,), jnp.float32),
        pltpu.VMEM_SHARED((_NP,), jnp.float32),
        pltpu.VMEM_SHARED((16, 16), jnp.float32),
    ],
)


@jax.jit
def kernel(x, edge_index):
    del x
    out = _pr_call(edge_index[0], edge_index[1])
    return out[:_N]
